# baseline (speedup 1.0000x reference)
import functools

import jax
import jax.numpy as jnp
from jax import lax
from jax.experimental import pallas as pl
from jax.experimental.pallas import tpu as pltpu
from jax.experimental.pallas import tpu_sc as plsc

N = 10000
E = 320000
D = 128
ALPHA = 0.9

NC = 2
NS = 16
NW = NC * NS
EPT = E // NW
CHUNK = 80
NCHUNK = EPT // CHUNK
NB = 12
NGROUP = -(-NCHUNK // NB)
RPN = N // NS
PD = 10240
RPD = PD // NS
DEGW = 8
DEG_DEPTH = 8
F1 = 32
F2 = 16
R = 2000
GRID = N // R

_BN3 = 1.0 / (1.0 + 1e-3) ** 0.5
_BN5 = 1.0 / (1.0 + 1e-5) ** 0.5
_QEXP = (ALPHA + 1.0) / 2.0


def _mesh():
    return plsc.VectorSubcoreMesh(core_axis_name="c", subcore_axis_name="s",
                                  num_cores=NC, num_subcores=NS)


def _build_sc_deg():
    @functools.partial(
        pl.kernel,
        out_type=jax.ShapeDtypeStruct((NC, PD, DEGW), jnp.float32),
        mesh=_mesh(),
        compiler_params=pltpu.CompilerParams(use_tc_tiling_on_sc=False),
        scratch_types=[
            pltpu.VMEM((EPT,), jnp.int32),
            pltpu.VMEM((CHUNK, DEGW), jnp.float32),
            pltpu.VMEM_SHARED((PD, DEGW), jnp.float32),
            pltpu.SemaphoreType.DMA,
        ],
    )
    def deg(ei_hbm, ones_hbm, zeros_hbm, out_hbm, didx, ones_v, acc, ssem):
        c = lax.axis_index("c")
        s = lax.axis_index("s")
        wid = s * NC + c
        pltpu.sync_copy(ones_hbm, ones_v)
        pltpu.sync_copy(ei_hbm.at[1, pl.ds(wid * EPT, EPT)], didx)
        pltpu.sync_copy(zeros_hbm.at[pl.ds(s * RPD, RPD)], acc.at[pl.ds(s * RPD, RPD)])
        plsc.subcore_barrier()

        def body(i, carry):
            idx = didx.at[pl.ds(i * CHUNK, CHUNK)]

            @pl.when(i >= DEG_DEPTH)
            def _():
                pltpu.make_async_copy(ones_v, acc.at[idx], ssem).wait()
            pltpu.async_copy(ones_v, acc.at[idx], ssem, add=True)
            return carry

        lax.fori_loop(0, NCHUNK, body, 0)

        def drain(i, carry):
            pltpu.make_async_copy(ones_v, acc.at[didx.at[pl.ds(0, CHUNK)]], ssem).wait()
            return carry

        lax.fori_loop(0, DEG_DEPTH, drain, 0)
        plsc.subcore_barrier()
        pltpu.sync_copy(acc.at[pl.ds(s * RPD, RPD)], out_hbm.at[c, pl.ds(s * RPD, RPD)])

    return deg


def _build_sc_prop(F):
    @functools.partial(
        pl.kernel,
        out_type=jax.ShapeDtypeStruct((NC, N, F), jnp.float32),
        mesh=_mesh(),
        compiler_params=pltpu.CompilerParams(use_tc_tiling_on_sc=False),
        scratch_types=[
            pltpu.VMEM((EPT,), jnp.int32),
            pltpu.VMEM((EPT,), jnp.int32),
            pltpu.VMEM((NB, CHUNK, F), jnp.float32),
            pltpu.VMEM_SHARED((N, F), jnp.float32),
            pltpu.SemaphoreType.DMA((NB,)),
            pltpu.SemaphoreType.DMA((NB,)),
        ],
    )
    def prop(ei_hbm, h_hbm, zeros_hbm, out_hbm, sidx, didx, rows, acc, gsem, ssem):
        c = lax.axis_index("c")
        s = lax.axis_index("s")
        wid = s * NC + c
        pltpu.sync_copy(ei_hbm.at[0, pl.ds(wid * EPT, EPT)], sidx)
        pltpu.sync_copy(ei_hbm.at[1, pl.ds(wid * EPT, EPT)], didx)
        pltpu.sync_copy(zeros_hbm.at[pl.ds(s * RPN, RPN)], acc.at[pl.ds(s * RPN, RPN)])
        plsc.subcore_barrier()

        def gth(i, b):
            return pltpu.make_async_copy(
                h_hbm.at[sidx.at[pl.ds(i * CHUNK, CHUNK)]], rows.at[b], gsem.at[b])

        def sct(i, b):
            return pltpu.make_async_copy(
                rows.at[b], acc.at[didx.at[pl.ds(i * CHUNK, CHUNK)]], ssem.at[b])

        for b in range(NB):
            gth(b, b).start()

        def group(g, carry):
            i0 = g * NB
            for b in range(NB):
                i = i0 + b

                @pl.when(i < NCHUNK)
                def _():
                    gth(i, b).wait()
                    pltpu.async_copy(rows.at[b],
                                     acc.at[didx.at[pl.ds(i * CHUNK, CHUNK)]],
                                     ssem.at[b], add=True)
            for b in range(NB):
                i = i0 + b
                i2 = i + NB

                @pl.when(i < NCHUNK)
                def _():
                    sct(i, b).wait()

                @pl.when(i2 < NCHUNK)
                def _():
                    gth(i2, b).start()
            return carry

        lax.fori_loop(0, NGROUP, group, 0)
        plsc.subcore_barrier()
        pltpu.sync_copy(acc.at[pl.ds(s * RPN, RPN)], out_hbm.at[c, pl.ds(s * RPN, RPN)])

    return prop


_SC_CACHE = {}


def _sc_deg(ei, ones8, z8):
    if 'deg' not in _SC_CACHE:
        _SC_CACHE['deg'] = _build_sc_deg()
    return _SC_CACHE['deg'](ei, ones8, z8)


def _sc_prop1(ei, h, zeros):
    if F1 not in _SC_CACHE:
        _SC_CACHE[F1] = _build_sc_prop(F1)
    return _SC_CACHE[F1](ei, h, zeros)


def _sc_prop2(ei, h, zeros):
    if F2 not in _SC_CACHE:
        _SC_CACHE[F2] = _build_sc_prop(F2)
    return _SC_CACHE[F2](ei, h, zeros)


def _elu(v):
    return jnp.where(v > 0, v, jnp.exp(jnp.minimum(v, 0.0)) - 1.0)


def _dinv_block(degp):
    deg = degp[0, :, 0:1] + degp[1, :, 0:1] + 1.0
    return lax.rsqrt(deg)


def _enc_body(x_ref, e0W, e0b, e0g, e0z, e1W, e1b, e1g, e1z, fx_ref):
    h = jnp.dot(x_ref[...], e0W[...], preferred_element_type=jnp.float32) + e0b[...]
    h = _elu(h * (e0g[...] * _BN3) + e0z[...])
    h = jnp.dot(h, e1W[...], preferred_element_type=jnp.float32) + e1b[...]
    fx_ref[...] = _elu(h * (e1g[...] * _BN3) + e1z[...])


def _scale_body(fx_ref, degp_ref, fxp_ref):
    dinv = _dinv_block(degp_ref[...])
    fxp_ref[...] = jnp.concatenate(
        [fx_ref[...] * dinv, jnp.zeros((R, F1 - 20), jnp.float32)], axis=1)


def _mid_body(p_ref, fxp_ref, degp_ref, cW, cb, cg, cz, Wml, h2p_ref):
    dinv = _dinv_block(degp_ref[...])
    p = p_ref[...]
    t = (p[0] + p[1] + fxp_ref[...]) * dinv
    s1 = jnp.dot(t, cW[...], preferred_element_type=jnp.float32) + cb[...]
    cc = jnp.maximum(s1 * (cg[...] * _BN5) + cz[...], 0.0)
    h2p_ref[...] = jnp.dot(cc, Wml[...], preferred_element_type=jnp.float32) * dinv


def _fin_body(r_ref, h2p_ref, degp_ref, mlb, fx_ref, d0W, d0b, d0g, d0z,
              d1W, d1b, cl, mu_ref, lv_ref, z_ref, de_ref, q_ref):
    dinv = _dinv_block(degp_ref[...])
    r = r_ref[...]
    m = (r[0] + r[1] + h2p_ref[...]) * dinv + mlb[...]
    mu_ref[...] = m[:, :8]
    lv_ref[...] = m[:, 8:]
    z = jnp.concatenate([fx_ref[...], m[:, :8]], axis=1)
    z_ref[...] = z
    dd = jnp.dot(z, d0W[...], preferred_element_type=jnp.float32) + d0b[...]
    dd = _elu(dd * (d0g[...] * _BN3) + d0z[...])
    de_ref[...] = jnp.dot(dd, d1W[...], preferred_element_type=jnp.float32) + d1b[...]
    clv = cl[...]
    zn = jnp.sum(z * z, axis=1, keepdims=True)
    cross = lax.dot_general(z, clv, (((1,), (1,)), ((), ())),
                            preferred_element_type=jnp.float32)
    cn = lax.dot_general(jnp.ones((1, 28), jnp.float32), clv * clv,
                         (((1,), (1,)), ((), ())),
                         preferred_element_type=jnp.float32)
    dist2 = zn - 2.0 * cross + cn
    t = 1.0 / (1.0 + dist2 / ALPHA)
    q = jnp.exp(_QEXP * jnp.log(t))
    colid = lax.broadcasted_iota(jnp.int32, (R, 16), 1)
    q = jnp.where(colid < 15, q, 0.0)
    q = q / jnp.sum(q, axis=1, keepdims=True)
    q_ref[...] = q[:, :15]


def _full(shape):
    nd = len(shape)
    return pl.BlockSpec(shape, lambda i, nd=nd: (0,) * nd)


def _rows(w):
    return pl.BlockSpec((R, w), lambda i: (i, 0))


_DEGP_SPEC = pl.BlockSpec((NC, R, DEGW), lambda i: (0, i, 0))


_TC_PARAMS = pltpu.CompilerParams(dimension_semantics=("parallel",))


def _make_tc_encoder(interpret=False):
    return pl.pallas_call(
        _enc_body,
        grid=(GRID,),
        in_specs=[
            _rows(D),
            _full((D, 32)), _full((1, 32)), _full((1, 32)), _full((1, 32)),
            _full((32, 20)), _full((1, 20)), _full((1, 20)), _full((1, 20)),
        ],
        out_specs=[_rows(20)],
        out_shape=[jax.ShapeDtypeStruct((N, 20), jnp.float32)],
        compiler_params=_TC_PARAMS,
        interpret=interpret,
    )


def _make_tc_scale(interpret=False):
    return pl.pallas_call(
        _scale_body,
        grid=(GRID,),
        in_specs=[_rows(20), _DEGP_SPEC],
        out_specs=[_rows(F1)],
        out_shape=[jax.ShapeDtypeStruct((N, F1), jnp.float32)],
        compiler_params=_TC_PARAMS,
        interpret=interpret,
    )


def _make_tc_mid(interpret=False):
    return pl.pallas_call(
        _mid_body,
        grid=(GRID,),
        in_specs=[
            pl.BlockSpec((NC, R, F1), lambda i: (0, i, 0)),
            _rows(F1), _DEGP_SPEC,
            _full((F1, 64)), _full((1, 64)), _full((1, 64)), _full((1, 64)),
            _full((64, 16)),
        ],
        out_specs=[_rows(16)],
        out_shape=[jax.ShapeDtypeStruct((N, 16), jnp.float32)],
        compiler_params=_TC_PARAMS,
        interpret=interpret,
    )


def _make_tc_final(interpret=False):
    return pl.pallas_call(
        _fin_body,
        grid=(GRID,),
        in_specs=[
            pl.BlockSpec((NC, R, F2), lambda i: (0, i, 0)),
            _rows(F2), _DEGP_SPEC,
            _full((1, 16)), _rows(20),
            _full((28, 32)), _full((1, 32)), _full((1, 32)), _full((1, 32)),
            _full((32, D)), _full((1, D)), _full((16, 28)),
        ],
        out_specs=[_rows(8), _rows(8), _rows(28), _rows(D), _rows(15)],
        out_shape=[jax.ShapeDtypeStruct((N, 8), jnp.float32),
                   jax.ShapeDtypeStruct((N, 8), jnp.float32),
                   jax.ShapeDtypeStruct((N, 28), jnp.float32),
                   jax.ShapeDtypeStruct((N, D), jnp.float32),
                   jax.ShapeDtypeStruct((N, 15), jnp.float32)],
        compiler_params=_TC_PARAMS,
        interpret=interpret,
    )


_tc_encoder = _make_tc_encoder()
_tc_scale = _make_tc_scale()
_tc_mid = _make_tc_mid()
_tc_final = _make_tc_final()


@jax.jit
def _run(x, edge_index, params):
    p = params
    row = lambda a: a[None, :]

    ones8 = jnp.ones((CHUNK, DEGW), jnp.float32)
    z8 = jnp.zeros((PD, DEGW), jnp.float32)
    zf1 = jnp.zeros((N, F1), jnp.float32)
    zf2 = jnp.zeros((N, F2), jnp.float32)

    degp = _sc_deg(edge_index, ones8, z8)

    fx, = _tc_encoder(
        x,
        p['enc0_W'], row(p['enc0_b']), row(p['enc0_g']), row(p['enc0_beta']),
        p['enc1_W'], row(p['enc1_b']), row(p['enc1_g']), row(p['enc1_beta']))
    fxp, = _tc_scale(fx, degp)

    p1 = _sc_prop1(edge_index, fxp, zf1)

    cW = jnp.pad(p['conv_W'], ((0, F1 - 20), (0, 0)))
    Wml = jnp.concatenate([p['mean_W'], p['lv_W']], axis=1)
    h2p, = _tc_mid(p1, fxp, degp,
                   cW, row(p['conv_b']), row(p['convbn_g']), row(p['convbn_beta']),
                   Wml)

    p2 = _sc_prop2(edge_index, h2p, zf2)

    mlb = row(jnp.concatenate([p['mean_b'], p['lv_b']]))
    clp = jnp.pad(p['cluster'], ((0, 1), (0, 0)))
    mu, logvar, z, de, q = _tc_final(
        p2, h2p, degp, mlb, fx,
        p['dec0_W'], row(p['dec0_b']), row(p['dec0_g']), row(p['dec0_beta']),
        p['dec1_W'], row(p['dec1_b']), clp)

    return z, mu, logvar, de, q, fx, mu


def kernel(x, edge_index, params):
    return _run(x, edge_index, params)


# speedup vs baseline: 1.0020x; 1.0020x over previous
import functools

import jax
import jax.numpy as jnp
from jax import lax
from jax.experimental import pallas as pl
from jax.experimental.pallas import tpu as pltpu
from jax.experimental.pallas import tpu_sc as plsc

N = 10000
E = 320000
D = 128
ALPHA = 0.9

NC = 2
NS = 16
NW = NC * NS
EPT = E // NW
CHUNK = 80
NCHUNK = EPT // CHUNK
NB = 8
NGROUP = -(-NCHUNK // NB)
RPN = N // NS
PD = 10240
RPD = PD // NS
DEGW = 8
DEG_DEPTH = 4
F1 = 32
F2 = 16
R = 2000
GRID = N // R

_BN3 = 1.0 / (1.0 + 1e-3) ** 0.5
_BN5 = 1.0 / (1.0 + 1e-5) ** 0.5
_QEXP = (ALPHA + 1.0) / 2.0


def _mesh():
    return plsc.VectorSubcoreMesh(core_axis_name="c", subcore_axis_name="s",
                                  num_cores=NC, num_subcores=NS)


def _build_sc_deg():
    @functools.partial(
        pl.kernel,
        out_type=jax.ShapeDtypeStruct((NC, PD, DEGW), jnp.float32),
        mesh=_mesh(),
        compiler_params=pltpu.CompilerParams(use_tc_tiling_on_sc=False),
        scratch_types=[
            pltpu.VMEM((EPT,), jnp.int32),
            pltpu.VMEM((CHUNK, DEGW), jnp.float32),
            pltpu.VMEM_SHARED((PD, DEGW), jnp.float32),
            pltpu.SemaphoreType.DMA,
        ],
    )
    def deg(ei_hbm, ones_hbm, zeros_hbm, out_hbm, didx, ones_v, acc, ssem):
        c = lax.axis_index("c")
        s = lax.axis_index("s")
        wid = s * NC + c
        pltpu.sync_copy(ones_hbm, ones_v)
        pltpu.sync_copy(ei_hbm.at[1, pl.ds(wid * EPT, EPT)], didx)
        pltpu.sync_copy(zeros_hbm.at[pl.ds(s * RPD, RPD)], acc.at[pl.ds(s * RPD, RPD)])
        plsc.subcore_barrier()

        def body(i, carry):
            idx = didx.at[pl.ds(i * CHUNK, CHUNK)]

            @pl.when(i >= DEG_DEPTH)
            def _():
                pltpu.make_async_copy(ones_v, acc.at[idx], ssem).wait()
            pltpu.async_copy(ones_v, acc.at[idx], ssem, add=True)
            return carry

        lax.fori_loop(0, NCHUNK, body, 0)

        def drain(i, carry):
            pltpu.make_async_copy(ones_v, acc.at[didx.at[pl.ds(0, CHUNK)]], ssem).wait()
            return carry

        lax.fori_loop(0, DEG_DEPTH, drain, 0)
        plsc.subcore_barrier()
        pltpu.sync_copy(acc.at[pl.ds(s * RPD, RPD)], out_hbm.at[c, pl.ds(s * RPD, RPD)])

    return deg


def _build_sc_prop(F):
    @functools.partial(
        pl.kernel,
        out_type=jax.ShapeDtypeStruct((NC, N, F), jnp.float32),
        mesh=_mesh(),
        compiler_params=pltpu.CompilerParams(use_tc_tiling_on_sc=False),
        scratch_types=[
            pltpu.VMEM((EPT,), jnp.int32),
            pltpu.VMEM((EPT,), jnp.int32),
            pltpu.VMEM((NB, CHUNK, F), jnp.float32),
            pltpu.VMEM_SHARED((N, F), jnp.float32),
            pltpu.SemaphoreType.DMA((NB,)),
            pltpu.SemaphoreType.DMA((NB,)),
        ],
    )
    def prop(ei_hbm, h_hbm, zeros_hbm, out_hbm, sidx, didx, rows, acc, gsem, ssem):
        c = lax.axis_index("c")
        s = lax.axis_index("s")
        wid = s * NC + c
        pltpu.sync_copy(ei_hbm.at[0, pl.ds(wid * EPT, EPT)], sidx)
        pltpu.sync_copy(ei_hbm.at[1, pl.ds(wid * EPT, EPT)], didx)
        pltpu.sync_copy(zeros_hbm.at[pl.ds(s * RPN, RPN)], acc.at[pl.ds(s * RPN, RPN)])
        plsc.subcore_barrier()

        def gth(i, b):
            return pltpu.make_async_copy(
                h_hbm.at[sidx.at[pl.ds(i * CHUNK, CHUNK)]], rows.at[b], gsem.at[b])

        def sct(i, b):
            return pltpu.make_async_copy(
                rows.at[b], acc.at[didx.at[pl.ds(i * CHUNK, CHUNK)]], ssem.at[b])

        for b in range(NB):
            gth(b, b).start()

        def group(g, carry):
            i0 = g * NB
            for b in range(NB):
                i = i0 + b

                @pl.when(i < NCHUNK)
                def _():
                    gth(i, b).wait()
                    pltpu.async_copy(rows.at[b],
                                     acc.at[didx.at[pl.ds(i * CHUNK, CHUNK)]],
                                     ssem.at[b], add=True)
            for b in range(NB):
                i = i0 + b
                i2 = i + NB

                @pl.when(i < NCHUNK)
                def _():
                    sct(i, b).wait()

                @pl.when(i2 < NCHUNK)
                def _():
                    gth(i2, b).start()
            return carry

        lax.fori_loop(0, NGROUP, group, 0)
        plsc.subcore_barrier()
        pltpu.sync_copy(acc.at[pl.ds(s * RPN, RPN)], out_hbm.at[c, pl.ds(s * RPN, RPN)])

    return prop


_SC_CACHE = {}


def _sc_deg(ei, ones8, z8):
    if 'deg' not in _SC_CACHE:
        _SC_CACHE['deg'] = _build_sc_deg()
    return _SC_CACHE['deg'](ei, ones8, z8)


def _sc_prop1(ei, h, zeros):
    if F1 not in _SC_CACHE:
        _SC_CACHE[F1] = _build_sc_prop(F1)
    return _SC_CACHE[F1](ei, h, zeros)


def _sc_prop2(ei, h, zeros):
    if F2 not in _SC_CACHE:
        _SC_CACHE[F2] = _build_sc_prop(F2)
    return _SC_CACHE[F2](ei, h, zeros)


def _elu(v):
    return jnp.where(v > 0, v, jnp.exp(jnp.minimum(v, 0.0)) - 1.0)


def _dinv_block(degp):
    deg = degp[0, :, 0:1] + degp[1, :, 0:1] + 1.0
    return lax.rsqrt(deg)


def _enc_body(x_ref, e0W, e0b, e0g, e0z, e1W, e1b, e1g, e1z, fx_ref):
    h = jnp.dot(x_ref[...], e0W[...], preferred_element_type=jnp.float32) + e0b[...]
    h = _elu(h * (e0g[...] * _BN3) + e0z[...])
    h = jnp.dot(h, e1W[...], preferred_element_type=jnp.float32) + e1b[...]
    fx_ref[...] = _elu(h * (e1g[...] * _BN3) + e1z[...])


def _scale_body(fx_ref, degp_ref, fxp_ref):
    dinv = _dinv_block(degp_ref[...])
    fxp_ref[...] = jnp.concatenate(
        [fx_ref[...] * dinv, jnp.zeros((R, F1 - 20), jnp.float32)], axis=1)


def _mid_body(p_ref, fxp_ref, degp_ref, cW, cb, cg, cz, Wml, h2p_ref):
    dinv = _dinv_block(degp_ref[...])
    p = p_ref[...]
    t = (p[0] + p[1] + fxp_ref[...]) * dinv
    s1 = jnp.dot(t, cW[...], preferred_element_type=jnp.float32) + cb[...]
    cc = jnp.maximum(s1 * (cg[...] * _BN5) + cz[...], 0.0)
    h2p_ref[...] = jnp.dot(cc, Wml[...], preferred_element_type=jnp.float32) * dinv


def _fin_body(r_ref, h2p_ref, degp_ref, mlb, fx_ref, d0W, d0b, d0g, d0z,
              d1W, d1b, cl, mu_ref, lv_ref, z_ref, de_ref, q_ref):
    dinv = _dinv_block(degp_ref[...])
    r = r_ref[...]
    m = (r[0] + r[1] + h2p_ref[...]) * dinv + mlb[...]
    mu_ref[...] = m[:, :8]
    lv_ref[...] = m[:, 8:]
    z = jnp.concatenate([fx_ref[...], m[:, :8]], axis=1)
    z_ref[...] = z
    dd = jnp.dot(z, d0W[...], preferred_element_type=jnp.float32) + d0b[...]
    dd = _elu(dd * (d0g[...] * _BN3) + d0z[...])
    de_ref[...] = jnp.dot(dd, d1W[...], preferred_element_type=jnp.float32) + d1b[...]
    clv = cl[...]
    zn = jnp.sum(z * z, axis=1, keepdims=True)
    cross = lax.dot_general(z, clv, (((1,), (1,)), ((), ())),
                            preferred_element_type=jnp.float32)
    cn = lax.dot_general(jnp.ones((1, 28), jnp.float32), clv * clv,
                         (((1,), (1,)), ((), ())),
                         preferred_element_type=jnp.float32)
    dist2 = zn - 2.0 * cross + cn
    t = 1.0 / (1.0 + dist2 / ALPHA)
    q = jnp.exp(_QEXP * jnp.log(t))
    colid = lax.broadcasted_iota(jnp.int32, (R, 16), 1)
    q = jnp.where(colid < 15, q, 0.0)
    q = q / jnp.sum(q, axis=1, keepdims=True)
    q_ref[...] = q[:, :15]


def _full(shape):
    nd = len(shape)
    return pl.BlockSpec(shape, lambda i, nd=nd: (0,) * nd)


def _rows(w):
    return pl.BlockSpec((R, w), lambda i: (i, 0))


_DEGP_SPEC = pl.BlockSpec((NC, R, DEGW), lambda i: (0, i, 0))


_TC_PARAMS = pltpu.CompilerParams(dimension_semantics=("parallel",))


def _make_tc_encoder(interpret=False):
    return pl.pallas_call(
        _enc_body,
        grid=(GRID,),
        in_specs=[
            _rows(D),
            _full((D, 32)), _full((1, 32)), _full((1, 32)), _full((1, 32)),
            _full((32, 20)), _full((1, 20)), _full((1, 20)), _full((1, 20)),
        ],
        out_specs=[_rows(20)],
        out_shape=[jax.ShapeDtypeStruct((N, 20), jnp.float32)],
        compiler_params=_TC_PARAMS,
        interpret=interpret,
    )


def _make_tc_scale(interpret=False):
    return pl.pallas_call(
        _scale_body,
        grid=(GRID,),
        in_specs=[_rows(20), _DEGP_SPEC],
        out_specs=[_rows(F1)],
        out_shape=[jax.ShapeDtypeStruct((N, F1), jnp.float32)],
        compiler_params=_TC_PARAMS,
        interpret=interpret,
    )


def _make_tc_mid(interpret=False):
    return pl.pallas_call(
        _mid_body,
        grid=(GRID,),
        in_specs=[
            pl.BlockSpec((NC, R, F1), lambda i: (0, i, 0)),
            _rows(F1), _DEGP_SPEC,
            _full((F1, 64)), _full((1, 64)), _full((1, 64)), _full((1, 64)),
            _full((64, 16)),
        ],
        out_specs=[_rows(16)],
        out_shape=[jax.ShapeDtypeStruct((N, 16), jnp.float32)],
        compiler_params=_TC_PARAMS,
        interpret=interpret,
    )


def _make_tc_final(interpret=False):
    return pl.pallas_call(
        _fin_body,
        grid=(GRID,),
        in_specs=[
            pl.BlockSpec((NC, R, F2), lambda i: (0, i, 0)),
            _rows(F2), _DEGP_SPEC,
            _full((1, 16)), _rows(20),
            _full((28, 32)), _full((1, 32)), _full((1, 32)), _full((1, 32)),
            _full((32, D)), _full((1, D)), _full((16, 28)),
        ],
        out_specs=[_rows(8), _rows(8), _rows(28), _rows(D), _rows(15)],
        out_shape=[jax.ShapeDtypeStruct((N, 8), jnp.float32),
                   jax.ShapeDtypeStruct((N, 8), jnp.float32),
                   jax.ShapeDtypeStruct((N, 28), jnp.float32),
                   jax.ShapeDtypeStruct((N, D), jnp.float32),
                   jax.ShapeDtypeStruct((N, 15), jnp.float32)],
        compiler_params=_TC_PARAMS,
        interpret=interpret,
    )


_tc_encoder = _make_tc_encoder()
_tc_scale = _make_tc_scale()
_tc_mid = _make_tc_mid()
_tc_final = _make_tc_final()


@jax.jit
def _run(x, edge_index, params):
    p = params
    row = lambda a: a[None, :]

    ones8 = jnp.ones((CHUNK, DEGW), jnp.float32)
    z8 = jnp.zeros((PD, DEGW), jnp.float32)
    zf1 = jnp.zeros((N, F1), jnp.float32)
    zf2 = jnp.zeros((N, F2), jnp.float32)

    degp = _sc_deg(edge_index, ones8, z8)

    fx, = _tc_encoder(
        x,
        p['enc0_W'], row(p['enc0_b']), row(p['enc0_g']), row(p['enc0_beta']),
        p['enc1_W'], row(p['enc1_b']), row(p['enc1_g']), row(p['enc1_beta']))
    fxp, = _tc_scale(fx, degp)

    p1 = _sc_prop1(edge_index, fxp, zf1)

    cW = jnp.pad(p['conv_W'], ((0, F1 - 20), (0, 0)))
    Wml = jnp.concatenate([p['mean_W'], p['lv_W']], axis=1)
    h2p, = _tc_mid(p1, fxp, degp,
                   cW, row(p['conv_b']), row(p['convbn_g']), row(p['convbn_beta']),
                   Wml)

    p2 = _sc_prop2(edge_index, h2p, zf2)

    mlb = row(jnp.concatenate([p['mean_b'], p['lv_b']]))
    clp = jnp.pad(p['cluster'], ((0, 1), (0, 0)))
    mu, logvar, z, de, q = _tc_final(
        p2, h2p, degp, mlb, fx,
        p['dec0_W'], row(p['dec0_b']), row(p['dec0_g']), row(p['dec0_beta']),
        p['dec1_W'], row(p['dec1_b']), clp)

    return z, mu, logvar, de, q, fx, mu


def kernel(x, edge_index, params):
    return _run(x, edge_index, params)
